# baseline (speedup 1.0000x reference)
import functools

import jax
import jax.numpy as jnp
from jax import lax
from jax.experimental import pallas as pl
from jax.experimental.pallas import tpu as pltpu
from jax.experimental.pallas import tpu_sc as plsc

NUM_CODES = 1024
DIM = 64
COMMIT_COST = 0.25
BATCHES_PER_STEP = 2
IDX_TILE = 8

NW = 32
SC_CH = 128


def _argmin_block(z_ref, w_ref, idx_ref, idxf_ref):
    w = w_ref[...]
    wsq = jnp.sum(w * w, axis=1)
    ones_b = jnp.ones((DIM, 128), jnp.float32)
    G = NUM_CODES // 128
    T = z_ref.shape[1]
    idxs = []
    for b in range(BATCHES_PER_STEP):
        z = z_ref[b]
        zw2 = jax.lax.dot_general(
            z * (-2.0), w, (((1,), (1,)), ((), ())),
            precision=jax.lax.Precision.DEFAULT,
        )
        zsqb = jax.lax.dot_general(
            z * z, ones_b, (((1,), (0,)), ((), ())),
            precision=jax.lax.Precision.DEFAULT,
        )
        pm = zsqb + zw2[:, 0:128] + wsq[None, 0:128]
        gidx = jnp.zeros(pm.shape, jnp.int32)
        for g in range(1, G):
            dg = zsqb + zw2[:, g * 128:(g + 1) * 128] + wsq[None, g * 128:(g + 1) * 128]
            mask = dg < pm
            pm = jnp.where(mask, dg, pm)
            gidx = jnp.where(mask, g, gidx)
        m = jnp.min(pm, axis=1, keepdims=True)
        lane = jax.lax.broadcasted_iota(jnp.int32, pm.shape, 1)
        ci = jnp.where(pm == m, gidx * 128 + lane, NUM_CODES)
        idx = jnp.min(ci, axis=1)
        row = pl.program_id(0) % (IDX_TILE // BATCHES_PER_STEP)
        idx_ref[row * BATCHES_PER_STEP + b, :] = idx
        idxs.append(idx)
    idxf_ref[pl.ds(pl.program_id(0) * (BATCHES_PER_STEP * T), BATCHES_PER_STEP * T)] = (
        jnp.concatenate(idxs)
    )


def _sc_gather_body(table_hbm, idx_hbm, out_hbm, idx_v, rows_v, sem):
    wid = lax.axis_index("s") * 2 + lax.axis_index("c")
    n_chunks = (out_hbm.shape[0] // NW) // SC_CH
    for c in range(n_chunks):
        base = (wid * n_chunks + c) * SC_CH
        pltpu.sync_copy(idx_hbm.at[pl.ds(base, SC_CH)], idx_v)
        pltpu.async_copy(table_hbm.at[idx_v], rows_v, sem).wait()
        pltpu.sync_copy(rows_v, out_hbm.at[pl.ds(base, SC_CH), :])


def _st_block(z_ref, zqp_ref, zq_ref, part_ref):
    total = jnp.zeros((), jnp.float32)
    for b in range(BATCHES_PER_STEP):
        z = z_ref[b]
        zq = zqp_ref[b][:, 0:DIM]
        zq_ref[b] = z + (zq - z)
        total = total + jnp.sum((zq - z) ** 2)
    part_ref[...] = jnp.full((128,), total, jnp.float32)


def kernel(z, codebook):
    B, T, D = z.shape
    n = B * T
    nsteps = B // BATCHES_PER_STEP
    idx, idx_flat = pl.pallas_call(
        _argmin_block,
        grid=(nsteps,),
        in_specs=[
            pl.BlockSpec((BATCHES_PER_STEP, T, D), lambda i: (i, 0, 0)),
            pl.BlockSpec((NUM_CODES, D), lambda i: (0, 0)),
        ],
        out_specs=[
            pl.BlockSpec(
                (IDX_TILE, T),
                lambda i: (i // (IDX_TILE // BATCHES_PER_STEP), 0),
            ),
            pl.BlockSpec((36864,), lambda i: (0,)),
        ],
        out_shape=[
            jax.ShapeDtypeStruct((B, T), jnp.int32),
            jax.ShapeDtypeStruct((n,), jnp.int32),
        ],
        compiler_params=pltpu.CompilerParams(
            dimension_semantics=("arbitrary",),
        ),
    )(z, codebook)

    table_pad = jnp.pad(codebook, ((0, 0), (0, 128 - DIM)))
    mesh = plsc.VectorSubcoreMesh(core_axis_name="c", subcore_axis_name="s")
    sc_gather = functools.partial(
        pl.kernel,
        mesh=mesh,
        out_type=jax.ShapeDtypeStruct((n, 128), jnp.float32),
        scratch_types=[
            pltpu.VMEM((SC_CH,), jnp.int32),
            pltpu.VMEM((SC_CH, 128), jnp.float32),
            pltpu.SemaphoreType.DMA,
        ],
    )(_sc_gather_body)
    zq_pad = sc_gather(table_pad, idx_flat)

    zq_st, parts = pl.pallas_call(
        _st_block,
        grid=(nsteps,),
        in_specs=[
            pl.BlockSpec((BATCHES_PER_STEP, T, D), lambda i: (i, 0, 0)),
            pl.BlockSpec((BATCHES_PER_STEP, T, 128), lambda i: (i, 0, 0)),
        ],
        out_specs=[
            pl.BlockSpec((BATCHES_PER_STEP, T, D), lambda i: (i, 0, 0)),
            pl.BlockSpec((128,), lambda i: (i,)),
        ],
        out_shape=[
            jax.ShapeDtypeStruct((B, T, D), jnp.float32),
            jax.ShapeDtypeStruct((nsteps * 128,), jnp.float32),
        ],
        compiler_params=pltpu.CompilerParams(
            dimension_semantics=("arbitrary",),
        ),
    )(z, zq_pad.reshape(B, T, 128))

    loss = (1.0 + COMMIT_COST) * jnp.sum(parts) / (128.0 * n * D)
    return (zq_st, loss, idx)


# speedup vs baseline: 1.0223x; 1.0223x over previous
import functools

import jax
import jax.numpy as jnp
from jax import lax
from jax.experimental import pallas as pl
from jax.experimental.pallas import tpu as pltpu
from jax.experimental.pallas import tpu_sc as plsc

NUM_CODES = 1024
DIM = 64
COMMIT_COST = 0.25
BATCHES_PER_STEP = 2
IDX_TILE = 8

NW = 32
SC_CH = 128


def _argmin_block(z_ref, w_ref, idx_ref, idxf_ref):
    w = w_ref[...]
    wsq = jnp.sum(w * w, axis=1)
    ones_b = jnp.ones((DIM, 128), jnp.float32)
    G = NUM_CODES // 128
    T = z_ref.shape[1]
    idxs = []
    for b in range(BATCHES_PER_STEP):
        z = z_ref[b]
        zw2 = jax.lax.dot_general(
            z * (-2.0), w, (((1,), (1,)), ((), ())),
            precision=jax.lax.Precision.DEFAULT,
        )
        zsqb = jax.lax.dot_general(
            z * z, ones_b, (((1,), (0,)), ((), ())),
            precision=jax.lax.Precision.DEFAULT,
        )
        pm = zsqb + zw2[:, 0:128] + wsq[None, 0:128]
        gidx = jnp.zeros(pm.shape, jnp.int32)
        for g in range(1, G):
            dg = zsqb + zw2[:, g * 128:(g + 1) * 128] + wsq[None, g * 128:(g + 1) * 128]
            mask = dg < pm
            pm = jnp.where(mask, dg, pm)
            gidx = jnp.where(mask, g, gidx)
        m = jnp.min(pm, axis=1, keepdims=True)
        lane = jax.lax.broadcasted_iota(jnp.int32, pm.shape, 1)
        ci = jnp.where(pm == m, gidx * 128 + lane, NUM_CODES)
        idx = jnp.min(ci, axis=1)
        row = pl.program_id(0) % (IDX_TILE // BATCHES_PER_STEP)
        idx_ref[row * BATCHES_PER_STEP + b, :] = idx
        idxs.append(idx)
    idxf_ref[pl.ds(pl.program_id(0) * (BATCHES_PER_STEP * T), BATCHES_PER_STEP * T)] = (
        jnp.concatenate(idxs)
    )


def _sc_gather_body(table_hbm, idx_hbm, out_hbm, idx_v, rows_v, sem):
    wid = lax.axis_index("s") * 2 + lax.axis_index("c")
    n_chunks = (out_hbm.shape[0] // NW) // SC_CH
    for c in range(n_chunks):
        base = (wid * n_chunks + c) * SC_CH
        pltpu.sync_copy(idx_hbm.at[pl.ds(base, SC_CH)], idx_v)
        pltpu.async_copy(table_hbm.at[idx_v], rows_v, sem).wait()
        pltpu.sync_copy(rows_v, out_hbm.at[pl.ds(base, SC_CH), :])


def _st_block(z_ref, zqp_ref, zq_ref, part_ref):
    total = jnp.zeros((), jnp.float32)
    for b in range(BATCHES_PER_STEP):
        z = z_ref[b]
        zq = zqp_ref[b][:, 0:DIM]
        zq_ref[b] = z + (zq - z)
        total = total + jnp.sum((zq - z) ** 2)
    part_ref[...] = jnp.full((128,), total, jnp.float32)


def _argmin_stage(zh, codebook):
    Bh, T, D = zh.shape
    nh = Bh * T
    nsteps = Bh // BATCHES_PER_STEP
    return pl.pallas_call(
        _argmin_block,
        grid=(nsteps,),
        in_specs=[
            pl.BlockSpec((BATCHES_PER_STEP, T, D), lambda i: (i, 0, 0)),
            pl.BlockSpec((NUM_CODES, D), lambda i: (0, 0)),
        ],
        out_specs=[
            pl.BlockSpec(
                (IDX_TILE, T),
                lambda i: (i // (IDX_TILE // BATCHES_PER_STEP), 0),
            ),
            pl.BlockSpec((nh,), lambda i: (0,)),
        ],
        out_shape=[
            jax.ShapeDtypeStruct((Bh, T), jnp.int32),
            jax.ShapeDtypeStruct((nh,), jnp.int32),
        ],
        compiler_params=pltpu.CompilerParams(
            dimension_semantics=("arbitrary",),
        ),
    )(zh, codebook)


def _st_stage(zh, zq_pad):
    Bh, T, D = zh.shape
    nsteps = Bh // BATCHES_PER_STEP
    return pl.pallas_call(
        _st_block,
        grid=(nsteps,),
        in_specs=[
            pl.BlockSpec((BATCHES_PER_STEP, T, D), lambda i: (i, 0, 0)),
            pl.BlockSpec((BATCHES_PER_STEP, T, 128), lambda i: (i, 0, 0)),
        ],
        out_specs=[
            pl.BlockSpec((BATCHES_PER_STEP, T, D), lambda i: (i, 0, 0)),
            pl.BlockSpec((128,), lambda i: (i,)),
        ],
        out_shape=[
            jax.ShapeDtypeStruct((Bh, T, D), jnp.float32),
            jax.ShapeDtypeStruct((nsteps * 128,), jnp.float32),
        ],
        compiler_params=pltpu.CompilerParams(
            dimension_semantics=("arbitrary",),
        ),
    )(zh, zq_pad.reshape(Bh, T, 128))


def kernel(z, codebook):
    B, T, D = z.shape
    n = B * T
    table_pad = jnp.pad(codebook, ((0, 0), (0, 128 - DIM)))
    mesh = plsc.VectorSubcoreMesh(core_axis_name="c", subcore_axis_name="s")
    halves = []
    for h in range(2):
        zh = lax.slice_in_dim(z, h * (B // 2), (h + 1) * (B // 2), axis=0)
        idx_h, idxf_h = _argmin_stage(zh, codebook)
        sc_gather = functools.partial(
            pl.kernel,
            mesh=mesh,
            out_type=jax.ShapeDtypeStruct((n // 2, 128), jnp.float32),
            scratch_types=[
                pltpu.VMEM((SC_CH,), jnp.int32),
                pltpu.VMEM((SC_CH, 128), jnp.float32),
                pltpu.SemaphoreType.DMA,
            ],
        )(_sc_gather_body)
        zq_pad_h = sc_gather(table_pad, idxf_h)
        halves.append((zh, idx_h, zq_pad_h))

    zq_parts = [_st_stage(zh, zq_pad_h) for zh, _, zq_pad_h in halves]
    zq_st = jnp.concatenate([p[0] for p in zq_parts], axis=0)
    idx = jnp.concatenate([h[1] for h in halves], axis=0)
    total = jnp.sum(zq_parts[0][1]) + jnp.sum(zq_parts[1][1])
    loss = (1.0 + COMMIT_COST) * total / (128.0 * n * D)
    return (zq_st, loss, idx)
